# baseline (speedup 1.0000x reference)
import jax
import jax.numpy as jnp
from jax.experimental import pallas as pl
from jax.experimental.pallas import tpu as pltpu

_N = 2048
_NUM_PHYS = 600000
_THRESH = 1.0
_BLK = 256
_HALF = _N // 2
_W = _HALF + _BLK
_CW = 128
_NCHUNK = _W // _CW
_NMASK = _BLK // _CW
_NSTRIP = _N // _BLK
_U = 2
_NGRP = _BLK // (8 * _U)


def _notch_kernel(gate_ref, xc_ref, yc_ref, hxc_ref, hyc_ref,
                  xr_ref, yr_ref, hxr_ref, hyr_ref, out_ref,
                  w_ref, acc_ref):
    r = pl.program_id(0)

    @pl.when(r == 0)
    def _():
        lrow = jax.lax.broadcasted_iota(jnp.int32, (_BLK, 2 * _BLK), 0)
        lcol = jax.lax.broadcasted_iota(jnp.int32, (_BLK, 2 * _BLK), 1)
        t = jnp.where(lcol < _BLK, lcol, lcol + (_HALF - _BLK)) - lrow
        w = jnp.where((t >= 1) & (t < _HALF), 1.0,
                      jnp.where(t == _HALF, 0.5, 0.0))
        w_ref[...] = w.astype(jnp.float32)
        acc_ref[...] = jnp.zeros((8, _CW), jnp.float32)

    base = r * _BLK

    def tile(off, k):
        co = base + k * _CW
        xc = xc_ref[pl.ds(off, 8), :]
        yc = yc_ref[pl.ds(off, 8), :]
        hxc = hxc_ref[pl.ds(off, 8), :]
        hyc = hyc_ref[pl.ds(off, 8), :]
        xr = xr_ref[:, pl.ds(co, _CW)]
        yr = yr_ref[:, pl.ds(co, _CW)]
        hxr = hxr_ref[:, pl.ds(co, _CW)]
        hyr = hyr_ref[:, pl.ds(co, _CW)]
        dx = jnp.maximum(jnp.abs(xc - xr) - (hxc + hxr), 0.0)
        dy = jnp.maximum(jnp.abs(yc - yr) - (hyc + hyr), 0.0)
        p = jnp.maximum((_THRESH - dx) - dy, 0.0)
        p2 = p * p
        if k < _NMASK:
            return w_ref[pl.ds(off, 8), k * _CW:(k + 1) * _CW] * p2
        if k >= _NCHUNK - _NMASK:
            kk = _NMASK + (k - (_NCHUNK - _NMASK))
            return w_ref[pl.ds(off, 8), kk * _CW:(kk + 1) * _CW] * p2
        return p2

    def body(g, accs):
        new = []
        for u in range(_U):
            a = accs[u]
            off = g * (8 * _U) + u * 8
            for k in range(_NCHUNK):
                a = a + tile(off, k)
            new.append(a)
        return tuple(new)

    zero = jnp.zeros((8, _CW), jnp.float32)
    accs = jax.lax.fori_loop(0, _NGRP, body, (zero,) * _U)
    acc_ref[...] += sum(accs[1:], accs[0])

    @pl.when(r == _NSTRIP - 1)
    def _():
        out_ref[0, 0] = jnp.sum(acc_ref[...]) * gate_ref[0, 0]


def kernel(pos, macro_mask, macro_size_x, macro_size_y):
    x = jax.lax.slice(pos, (0,), (_N,))
    y = jax.lax.slice(pos, (_NUM_PHYS,), (_NUM_PHYS + _N,))
    m = macro_mask
    neg = jnp.where(m, jnp.float32(0.0), jnp.float32(-1e7))
    hx = macro_size_x.astype(jnp.float32) * 0.5 + neg
    hy = macro_size_y.astype(jnp.float32) * 0.5 + neg
    count = jnp.sum(m.astype(jnp.int32))
    gate = jnp.where(count < 2, 0.0, 1.0).astype(jnp.float32).reshape(1, 1)

    col = lambda v: v.reshape(_N, 1)
    dbl = lambda v: jnp.concatenate([v, v]).reshape(1, 2 * _N)

    out = pl.pallas_call(
        _notch_kernel,
        grid=(_NSTRIP,),
        in_specs=[
            pl.BlockSpec(memory_space=pltpu.SMEM),
            pl.BlockSpec((_BLK, 1), lambda r: (r, 0)),
            pl.BlockSpec((_BLK, 1), lambda r: (r, 0)),
            pl.BlockSpec((_BLK, 1), lambda r: (r, 0)),
            pl.BlockSpec((_BLK, 1), lambda r: (r, 0)),
            pl.BlockSpec((1, 2 * _N), lambda r: (0, 0)),
            pl.BlockSpec((1, 2 * _N), lambda r: (0, 0)),
            pl.BlockSpec((1, 2 * _N), lambda r: (0, 0)),
            pl.BlockSpec((1, 2 * _N), lambda r: (0, 0)),
        ],
        out_shape=jax.ShapeDtypeStruct((1, 1), jnp.float32),
        out_specs=pl.BlockSpec(memory_space=pltpu.SMEM),
        scratch_shapes=[
            pltpu.VMEM((_BLK, 2 * _BLK), jnp.float32),
            pltpu.VMEM((8, _CW), jnp.float32),
        ],
        compiler_params=pltpu.CompilerParams(
            dimension_semantics=("arbitrary",)),
    )(gate, col(x), col(y), col(hx), col(hy), dbl(x), dbl(y), dbl(hx), dbl(hy))

    return out.reshape(())


# speedup vs baseline: 1.4991x; 1.4991x over previous
import jax
import jax.numpy as jnp
from jax.experimental import pallas as pl
from jax.experimental.pallas import tpu as pltpu

_N = 2048
_NUM_PHYS = 600000
_THRESH = 1.0
_BLK = 256
_HALF = _N // 2
_MID = _HALF - _BLK
_NSTRIP = _N // _BLK


def _notch_kernel(gate_ref, axc_ref, bxc_ref, ayc_ref, byc_ref,
                  axr_ref, bxr_ref, ayr_ref, byr_ref, out_ref):
    r = pl.program_id(0)
    base = r * _BLK
    axc = axc_ref[...]
    bxc = bxc_ref[...]
    ayc = ayc_ref[...]
    byc = byc_ref[...]

    def p2(co, w):
        axr = axr_ref[:, pl.ds(co, w)]
        bxr = bxr_ref[:, pl.ds(co, w)]
        ayr = ayr_ref[:, pl.ds(co, w)]
        byr = byr_ref[:, pl.ds(co, w)]
        dx = jnp.maximum(jnp.maximum(axc - bxr, axr - bxc), 0.0)
        dy = jnp.maximum(jnp.maximum(ayc - byr, ayr - byc), 0.0)
        p = jnp.maximum((_THRESH - dx) - dy, 0.0)
        return p * p

    lrow = jax.lax.broadcasted_iota(jnp.int32, (_BLK, _BLK), 0)
    lcol = jax.lax.broadcasted_iota(jnp.int32, (_BLK, _BLK), 1)

    s = jnp.sum(jnp.where(lcol > lrow, p2(base, _BLK), 0.0))
    s += jnp.sum(p2(base + _BLK, _MID))
    wlast = jnp.where(lcol < lrow, 1.0,
                      jnp.where(lcol == lrow, 0.5, 0.0)).astype(jnp.float32)
    s += jnp.sum(wlast * p2(base + _HALF, _BLK))

    @pl.when(r == 0)
    def _():
        out_ref[0, 0] = 0.0

    out_ref[0, 0] += s

    @pl.when(r == _NSTRIP - 1)
    def _():
        out_ref[0, 0] = out_ref[0, 0] * gate_ref[0, 0]


def kernel(pos, macro_mask, macro_size_x, macro_size_y):
    x = jax.lax.slice(pos, (0,), (_N,))
    y = jax.lax.slice(pos, (_NUM_PHYS,), (_NUM_PHYS + _N,))
    m = macro_mask
    neg = jnp.where(m, jnp.float32(0.0), jnp.float32(-1e7))
    hx = macro_size_x.astype(jnp.float32) * 0.5 + neg
    hy = macro_size_y.astype(jnp.float32) * 0.5 + neg
    ax, bx = x - hx, x + hx
    ay, by = y - hy, y + hy
    count = jnp.sum(m.astype(jnp.int32))
    gate = jnp.where(count < 2, 0.0, 1.0).astype(jnp.float32).reshape(1, 1)

    col = lambda v: v.reshape(_N, 1)
    dbl = lambda v: jnp.concatenate([v, v]).reshape(1, 2 * _N)

    out = pl.pallas_call(
        _notch_kernel,
        grid=(_NSTRIP,),
        in_specs=[
            pl.BlockSpec(memory_space=pltpu.SMEM),
            pl.BlockSpec((_BLK, 1), lambda r: (r, 0)),
            pl.BlockSpec((_BLK, 1), lambda r: (r, 0)),
            pl.BlockSpec((_BLK, 1), lambda r: (r, 0)),
            pl.BlockSpec((_BLK, 1), lambda r: (r, 0)),
            pl.BlockSpec((1, 2 * _N), lambda r: (0, 0)),
            pl.BlockSpec((1, 2 * _N), lambda r: (0, 0)),
            pl.BlockSpec((1, 2 * _N), lambda r: (0, 0)),
            pl.BlockSpec((1, 2 * _N), lambda r: (0, 0)),
        ],
        out_shape=jax.ShapeDtypeStruct((1, 1), jnp.float32),
        out_specs=pl.BlockSpec(memory_space=pltpu.SMEM),
        compiler_params=pltpu.CompilerParams(
            dimension_semantics=("arbitrary",)),
    )(gate, col(ax), col(bx), col(ay), col(by),
      dbl(ax), dbl(bx), dbl(ay), dbl(by))

    return out.reshape(())


# speedup vs baseline: 1.5194x; 1.0135x over previous
import jax
import jax.numpy as jnp
from jax.experimental import pallas as pl
from jax.experimental.pallas import tpu as pltpu

_N = 2048
_NUM_PHYS = 600000
_THRESH = 1.0
_BLK = 256
_HALF = _N // 2
_MID = _HALF - _BLK
_NSTRIP = _N // _BLK


def _notch_kernel(gate_ref, axc_ref, bxc_ref, ayc_ref, byc_ref,
                  axr_ref, bxr_ref, ayr_ref, byr_ref, out_ref):
    lrow = jax.lax.broadcasted_iota(jnp.int32, (_BLK, _BLK), 0)
    lcol = jax.lax.broadcasted_iota(jnp.int32, (_BLK, _BLK), 1)
    upper = lcol > lrow
    wlast = jnp.where(lcol < lrow, 1.0,
                      jnp.where(lcol == lrow, 0.5, 0.0)).astype(jnp.float32)

    def strip(r, acc):
        base = r * _BLK
        axc = axc_ref[pl.ds(base, _BLK), :]
        bxc = bxc_ref[pl.ds(base, _BLK), :]
        ayc = ayc_ref[pl.ds(base, _BLK), :]
        byc = byc_ref[pl.ds(base, _BLK), :]

        def p2(co, w):
            axr = axr_ref[:, pl.ds(co, w)]
            bxr = bxr_ref[:, pl.ds(co, w)]
            ayr = ayr_ref[:, pl.ds(co, w)]
            byr = byr_ref[:, pl.ds(co, w)]
            dx = jnp.maximum(jnp.maximum(axc - bxr, axr - bxc), 0.0)
            dy = jnp.maximum(jnp.maximum(ayc - byr, ayr - byc), 0.0)
            p = jnp.maximum((_THRESH - dx) - dy, 0.0)
            return p * p

        s = jnp.sum(jnp.where(upper, p2(base, _BLK), 0.0))
        s += jnp.sum(p2(base + _BLK, _MID))
        s += jnp.sum(wlast * p2(base + _HALF, _BLK))
        return acc + s

    total = jax.lax.fori_loop(0, _NSTRIP, strip, jnp.float32(0.0))
    out_ref[0, 0] = total * gate_ref[0, 0]


def kernel(pos, macro_mask, macro_size_x, macro_size_y):
    x = jax.lax.slice(pos, (0,), (_N,))
    y = jax.lax.slice(pos, (_NUM_PHYS,), (_NUM_PHYS + _N,))
    m = macro_mask
    neg = jnp.where(m, jnp.float32(0.0), jnp.float32(-1e7))
    hx = macro_size_x.astype(jnp.float32) * 0.5 + neg
    hy = macro_size_y.astype(jnp.float32) * 0.5 + neg
    ax, bx = x - hx, x + hx
    ay, by = y - hy, y + hy
    count = jnp.sum(m.astype(jnp.int32))
    gate = jnp.where(count < 2, 0.0, 1.0).astype(jnp.float32).reshape(1, 1)

    col = lambda v: v.reshape(_N, 1)
    dbl = lambda v: jnp.concatenate([v, v]).reshape(1, 2 * _N)

    out = pl.pallas_call(
        _notch_kernel,
        in_specs=[
            pl.BlockSpec(memory_space=pltpu.SMEM),
            pl.BlockSpec((_N, 1), lambda: (0, 0)),
            pl.BlockSpec((_N, 1), lambda: (0, 0)),
            pl.BlockSpec((_N, 1), lambda: (0, 0)),
            pl.BlockSpec((_N, 1), lambda: (0, 0)),
            pl.BlockSpec((1, 2 * _N), lambda: (0, 0)),
            pl.BlockSpec((1, 2 * _N), lambda: (0, 0)),
            pl.BlockSpec((1, 2 * _N), lambda: (0, 0)),
            pl.BlockSpec((1, 2 * _N), lambda: (0, 0)),
        ],
        out_shape=jax.ShapeDtypeStruct((1, 1), jnp.float32),
        out_specs=pl.BlockSpec(memory_space=pltpu.SMEM),
    )(gate, col(ax), col(bx), col(ay), col(by),
      dbl(ax), dbl(bx), dbl(ay), dbl(by))

    return out.reshape(())
